# baseline (speedup 1.0000x reference)
import functools

import jax
import jax.numpy as jnp
from jax import lax
from jax.experimental import pallas as pl
from jax.experimental.pallas import tpu as pltpu
from jax.experimental.pallas import tpu_sc as plsc

N = 10000
E = 320000
H = 128
NL = 6
OUT_DIM = 15

_NC = 2
_NS = 16
_NW = _NC * _NS
_C = 40
_EPW = E // _NW
_CHUNKS = _EPW // _C
_NPAD = 10240
_RPT = _NPAD // _NS

@functools.cache
def _make_sc_edge_pass():
    mesh = plsc.VectorSubcoreMesh(
        core_axis_name="c", subcore_axis_name="s",
        num_cores=_NC, num_subcores=_NS)

    buf = lambda: [pltpu.VMEM((_C,), jnp.int32),
                   pltpu.VMEM((_C,), jnp.int32),
                   pltpu.VMEM((_C, H), jnp.float32),
                   pltpu.VMEM((_C, H), jnp.float32),
                   pltpu.VMEM((_C, H), jnp.float32),
                   pltpu.VMEM((_C,), jnp.int32),
                   pltpu.SemaphoreType.DMA,
                   pltpu.SemaphoreType.DMA,
                   pltpu.SemaphoreType.DMA]

    @functools.partial(
        pl.kernel,
        out_type=jax.ShapeDtypeStruct((_NC, _NPAD, H), jnp.float32),
        mesh=mesh,
        scratch_types=buf() + buf() + [
            pltpu.VMEM_SHARED((_NPAD, H), jnp.float32),
        ],
    )
    def _sc_edge_pass(a_hbm, b_hbm, ce_hbm, src_hbm, dst_hbm, out_hbm,
                      src0, dst0, a0, b0, c0, ds0, si0, sg0, ss0,
                      src1, dst1, a1, b1, c1, ds1, si1, sg1, ss1, acc_sh):
        cid = lax.axis_index("c")
        sid = lax.axis_index("s")
        wid = sid * _NC + cid
        zero = jnp.zeros((16,), jnp.float32)
        bufs = ((src0, dst0, a0, b0, c0, ds0, si0, sg0, ss0),
                (src1, dst1, a1, b1, c1, ds1, si1, sg1, ss1))

        def zrow(i, carry):
            for j in range(8):
                a0[i, pl.ds(j * 16, 16)] = zero
            return carry
        lax.fori_loop(0, _C, zrow, 0)
        for k in range(_RPT // _C):
            pltpu.sync_copy(a0, acc_sh.at[pl.ds(sid * _RPT + k * _C, _C)])
        plsc.subcore_barrier()

        def idx_issue(t, b):
            src_v, dst_v = b[0], b[1]
            sem_i = b[6]
            base = wid * _EPW + t * _C
            pltpu.async_copy(src_hbm.at[pl.ds(base, _C)], src_v, sem_i)
            pltpu.async_copy(dst_hbm.at[pl.ds(base, _C)], dst_v, sem_i)

        def idx_drain(b):
            src_v, dst_v = b[0], b[1]
            sem_i = b[6]
            pltpu.make_async_copy(
                src_hbm.at[pl.ds(0, _C)], src_v, sem_i).wait()
            pltpu.make_async_copy(
                dst_hbm.at[pl.ds(0, _C)], dst_v, sem_i).wait()

        def gat_issue(t, b):
            src_v, dst_v, a_v, b_v, c_v = b[:5]
            sem_g = b[7]
            base = wid * _EPW + t * _C
            pltpu.async_copy(a_hbm.at[src_v], a_v, sem_g)
            pltpu.async_copy(b_hbm.at[dst_v], b_v, sem_g)
            pltpu.async_copy(ce_hbm.at[pl.ds(base, _C)], c_v, sem_g)

        def gat_drain(b):
            src_v, dst_v, a_v, b_v, c_v = b[:5]
            sem_g = b[7]
            pltpu.make_async_copy(a_hbm.at[src_v], a_v, sem_g).wait()
            pltpu.make_async_copy(b_hbm.at[dst_v], b_v, sem_g).wait()
            pltpu.make_async_copy(ce_hbm.at[pl.ds(0, _C)], c_v, sem_g).wait()

        def compute(b):
            dst_v, a_v, b_v, c_v, dst_s = b[1], b[2], b[3], b[4], b[5]

            def erow(i, cc):
                for r in range(2):
                    for j in range(8):
                        sl = pl.ds(j * 16, 16)
                        a_v[2 * i + r, sl] = jnp.maximum(
                            a_v[2 * i + r, sl] + b_v[2 * i + r, sl]
                            + c_v[2 * i + r, sl], 0.0)
                return cc
            lax.fori_loop(0, _C // 2, erow, 0)

        def body(t, p, tail=False, issue_idx=True):
            gat_drain(bufs[p])
            if not tail:
                idx_drain(bufs[1 - p])
                gat_issue(t + 1, bufs[1 - p])
            compute(bufs[p])
            pltpu.sync_copy(bufs[p][2], acc_sh.at[bufs[p][1]], add=True)
            if not tail and issue_idx:
                idx_issue(t + 2, bufs[p])

        idx_issue(0, bufs[0])
        idx_drain(bufs[0])
        gat_issue(0, bufs[0])
        idx_issue(1, bufs[1])

        def pair(q, carry):
            body(2 * q, 0)
            body(2 * q + 1, 1)
            return carry
        lax.fori_loop(0, (_CHUNKS - 2) // 2, pair, 0)

        body(_CHUNKS - 2, 0, issue_idx=False)
        body(_CHUNKS - 1, 1, tail=True)

        plsc.subcore_barrier()
        pltpu.sync_copy(acc_sh.at[pl.ds(sid * _RPT, _RPT)],
                        out_hbm.at[cid, pl.ds(sid * _RPT, _RPT)])

    return _sc_edge_pass


def _enc_nodes_body(x_ref, w0, b0, w1, b1, o_ref):
    h1 = jnp.maximum(jnp.dot(x_ref[...], w0[...],
                             preferred_element_type=jnp.float32) + b0[...], 0.0)
    o_ref[...] = jnp.dot(h1, w1[...],
                         preferred_element_type=jnp.float32) + b1[...]


def _enc_edges_body(ea_ref, w0, b0, w1, b1, wee, bee, o_ref):
    e1 = jnp.maximum(jnp.dot(ea_ref[...], w0[...],
                             preferred_element_type=jnp.float32) + b0[...], 0.0)
    e = jnp.dot(e1, w1[...], preferred_element_type=jnp.float32) + b1[...]
    for l in range(NL):
        o_ref[l] = jnp.dot(e, wee[l],
                           preferred_element_type=jnp.float32) + bee[l]


def _pre_body(h_ref, ws, wd, a_ref, b_ref):
    h = h_ref[...]
    a_ref[...] = jnp.dot(h, ws[...], preferred_element_type=jnp.float32)
    b_ref[...] = jnp.dot(h, wd[...], preferred_element_type=jnp.float32)


def _upd_body(h_ref, ag_ref, wnh, wna, bn, o_ref):
    h = h_ref[...]
    ag = ag_ref[0] + ag_ref[1]
    o_ref[...] = (h + jnp.dot(h, wnh[...], preferred_element_type=jnp.float32)
                  + jnp.dot(ag, wna[...], preferred_element_type=jnp.float32)
                  + bn[...])


def _updpre_body(h_ref, ag_ref, wnh, wna, bn, ws, wd, h_ref_o, a_ref, b_ref):
    h = h_ref[...]
    ag = ag_ref[0] + ag_ref[1]
    h2 = (h + jnp.dot(h, wnh[...], preferred_element_type=jnp.float32)
          + jnp.dot(ag, wna[...], preferred_element_type=jnp.float32)
          + bn[...])
    h_ref_o[...] = h2
    a_ref[...] = jnp.dot(h2, ws[...], preferred_element_type=jnp.float32)
    b_ref[...] = jnp.dot(h2, wd[...], preferred_element_type=jnp.float32)


def _dec_body(h_ref, w0, b0, w1, b1, w2, b2, m_ref, o_ref):
    z = jnp.maximum(jnp.dot(h_ref[...], w0[...],
                            preferred_element_type=jnp.float32) + b0[...], 0.0)
    z = jnp.maximum(jnp.dot(z, w1[...],
                            preferred_element_type=jnp.float32) + b1[...], 0.0)
    p = jnp.dot(z, w2[...], preferred_element_type=jnp.float32) + b2[...]
    o_ref[...] = p * m_ref[...]


def _full(shape):
    return pl.BlockSpec(shape, lambda i: tuple(0 for _ in shape))


def kernel(x, edge_index, edge_attr, bc_disp, bc_rot, face_mask,
           enc_W0, enc_b0, enc_W1, enc_b1,
           eenc_W0, eenc_b0, eenc_W1, eenc_b1,
           mp_We, mp_be, mp_Wn, mp_bn,
           dec_W0, dec_b0, dec_W1, dec_b1, dec_W2, dec_b2):
    f32 = jnp.float32
    blk_n = 2000
    blk_e = 1000

    xp = jnp.pad(x, ((0, 0), (0, 16 - x.shape[1])))
    enc_W0p = jnp.pad(enc_W0, ((0, 16 - enc_W0.shape[0]), (0, 0)))
    eap = jnp.pad(edge_attr, ((0, 0), (0, 16 - edge_attr.shape[1])))
    eenc_W0p = jnp.pad(eenc_W0, ((0, 16 - eenc_W0.shape[0]), (0, 0)))
    We_s = mp_We[:, :H, :]
    We_d = mp_We[:, H:2 * H, :]
    We_e = mp_We[:, 2 * H:, :]
    Wn_h = mp_Wn[:, :H, :]
    Wn_a = mp_Wn[:, H:, :]
    src1 = edge_index[0]
    dst1 = edge_index[1]
    dec_W2p = jnp.pad(dec_W2, ((0, 0), (0, 16 - OUT_DIM)))
    dec_b2p = jnp.pad(dec_b2, ((0, 16 - OUT_DIM),))
    disp_m = 1.0 - bc_disp
    rot_m = 1.0 - bc_rot
    force_m = jnp.repeat(face_mask, 3, axis=1)
    mask16 = jnp.concatenate(
        [disp_m, disp_m, rot_m, force_m, jnp.zeros((N, 1), f32)], axis=1)

    h = pl.pallas_call(
        _enc_nodes_body,
        grid=(N // blk_n,),
        in_specs=[pl.BlockSpec((blk_n, 16), lambda i: (i, 0)),
                  _full((16, H)), _full((1, H)), _full((H, H)), _full((1, H))],
        out_specs=pl.BlockSpec((blk_n, H), lambda i: (i, 0)),
        out_shape=jax.ShapeDtypeStruct((N, H), f32),
    )(xp, enc_W0p, enc_b0.reshape(1, H), enc_W1, enc_b1.reshape(1, H))

    ce = pl.pallas_call(
        _enc_edges_body,
        grid=(E // blk_e,),
        in_specs=[pl.BlockSpec((blk_e, 16), lambda i: (i, 0)),
                  _full((16, H)), _full((1, H)), _full((H, H)), _full((1, H)),
                  _full((NL, H, H)), _full((NL, 1, H))],
        out_specs=pl.BlockSpec((NL, blk_e, H), lambda i: (0, i, 0)),
        out_shape=jax.ShapeDtypeStruct((NL, E, H), f32),
    )(eap, eenc_W0p, eenc_b0.reshape(1, H), eenc_W1, eenc_b1.reshape(1, H),
      We_e, mp_be.reshape(NL, 1, H))
    ces = [ce[l] for l in range(NL)]

    a_n, b_n = pl.pallas_call(
        _pre_body,
        grid=(N // blk_n,),
        in_specs=[pl.BlockSpec((blk_n, H), lambda i: (i, 0)),
                  _full((H, H)), _full((H, H))],
        out_specs=[pl.BlockSpec((blk_n, H), lambda i: (i, 0)),
                   pl.BlockSpec((blk_n, H), lambda i: (i, 0))],
        out_shape=[jax.ShapeDtypeStruct((N, H), f32),
                   jax.ShapeDtypeStruct((N, H), f32)],
    )(h, We_s[0], We_d[0])

    for l in range(NL):
        aggp = _make_sc_edge_pass()(a_n, b_n, ces[l], src1, dst1)

        if l < NL - 1:
            h, a_n, b_n = pl.pallas_call(
                _updpre_body,
                grid=(N // blk_n,),
                in_specs=[pl.BlockSpec((blk_n, H), lambda i: (i, 0)),
                          pl.BlockSpec((_NC, blk_n, H), lambda i: (0, i, 0)),
                          _full((H, H)), _full((H, H)), _full((1, H)),
                          _full((H, H)), _full((H, H))],
                out_specs=[pl.BlockSpec((blk_n, H), lambda i: (i, 0)),
                           pl.BlockSpec((blk_n, H), lambda i: (i, 0)),
                           pl.BlockSpec((blk_n, H), lambda i: (i, 0))],
                out_shape=[jax.ShapeDtypeStruct((N, H), f32),
                           jax.ShapeDtypeStruct((N, H), f32),
                           jax.ShapeDtypeStruct((N, H), f32)],
            )(h, aggp, Wn_h[l], Wn_a[l], mp_bn[l].reshape(1, H),
              We_s[l + 1], We_d[l + 1])
        else:
            h = pl.pallas_call(
                _upd_body,
                grid=(N // blk_n,),
                in_specs=[pl.BlockSpec((blk_n, H), lambda i: (i, 0)),
                          pl.BlockSpec((_NC, blk_n, H), lambda i: (0, i, 0)),
                          _full((H, H)), _full((H, H)), _full((1, H))],
                out_specs=pl.BlockSpec((blk_n, H), lambda i: (i, 0)),
                out_shape=jax.ShapeDtypeStruct((N, H), f32),
            )(h, aggp, Wn_h[l], Wn_a[l], mp_bn[l].reshape(1, H))

    out16 = pl.pallas_call(
        _dec_body,
        grid=(N // blk_n,),
        in_specs=[pl.BlockSpec((blk_n, H), lambda i: (i, 0)),
                  _full((H, H)), _full((1, H)), _full((H, 64)), _full((1, 64)),
                  _full((64, 16)), _full((1, 16)),
                  pl.BlockSpec((blk_n, 16), lambda i: (i, 0))],
        out_specs=pl.BlockSpec((blk_n, 16), lambda i: (i, 0)),
        out_shape=jax.ShapeDtypeStruct((N, 16), f32),
    )(h, dec_W0, dec_b0.reshape(1, H), dec_W1, dec_b1.reshape(1, 64),
      dec_W2p, dec_b2p.reshape(1, 16), mask16)

    return out16[:, :OUT_DIM]


# speedup vs baseline: 1.2019x; 1.2019x over previous
import functools

import jax
import jax.numpy as jnp
from jax import lax
from jax.experimental import pallas as pl
from jax.experimental.pallas import tpu as pltpu
from jax.experimental.pallas import tpu_sc as plsc

N = 10000
E = 320000
H = 128
NL = 6
OUT_DIM = 15

_NC = 2
_NS = 16
_NW = _NC * _NS
_C = 40
_EPW = E // _NW
_CHUNKS = _EPW // _C
_NPAD = 10240
_RPT = _NPAD // _NS

@functools.cache
def _make_sc_edge_pass():
    mesh = plsc.VectorSubcoreMesh(
        core_axis_name="c", subcore_axis_name="s",
        num_cores=_NC, num_subcores=_NS)

    buf = lambda: [pltpu.VMEM((_C,), jnp.int32),
                   pltpu.VMEM((_C,), jnp.int32),
                   pltpu.VMEM((_C, H), jnp.float32),
                   pltpu.VMEM((_C, H), jnp.float32),
                   pltpu.VMEM((_C, H), jnp.float32),
                   pltpu.SemaphoreType.DMA,
                   pltpu.SemaphoreType.DMA]

    @functools.partial(
        pl.kernel,
        out_type=jax.ShapeDtypeStruct((_NC, _NPAD, H), jnp.float32),
        mesh=mesh,
        scratch_types=buf() + buf() + [
            pltpu.VMEM((2 * _C, H), jnp.float32),
            pltpu.VMEM((2 * _C,), jnp.int32),
            pltpu.VMEM_SHARED((_NPAD, H), jnp.float32),
        ],
    )
    def _sc_edge_pass(a_hbm, b_hbm, ce_hbm, src_hbm, dst_hbm, out_hbm,
                      src0, dst0, a0, b0, c0, si0, sg0,
                      src1, dst1, a1, b1, c1, si1, sg1, m_v, dstc, acc_sh):
        cid = lax.axis_index("c")
        sid = lax.axis_index("s")
        wid = sid * _NC + cid
        zero = jnp.zeros((16,), jnp.float32)
        bufs = ((src0, dst0, a0, b0, c0, si0, sg0),
                (src1, dst1, a1, b1, c1, si1, sg1))

        def idx_issue(t, b):
            src_v, dst_v, sem_i = b[0], b[1], b[5]
            base = wid * _EPW + t * _C
            pltpu.async_copy(src_hbm.at[pl.ds(base, _C)], src_v, sem_i)
            pltpu.async_copy(dst_hbm.at[pl.ds(base, _C)], dst_v, sem_i)

        def idx_drain(b):
            src_v, dst_v, sem_i = b[0], b[1], b[5]
            pltpu.make_async_copy(
                src_hbm.at[pl.ds(0, _C)], src_v, sem_i).wait()
            pltpu.make_async_copy(
                dst_hbm.at[pl.ds(0, _C)], dst_v, sem_i).wait()

        def gat_issue(t, b):
            src_v, dst_v, a_v, b_v, c_v = b[:5]
            sem_g = b[6]
            base = wid * _EPW + t * _C
            pltpu.async_copy(a_hbm.at[src_v], a_v, sem_g)
            pltpu.async_copy(b_hbm.at[dst_v], b_v, sem_g)
            pltpu.async_copy(ce_hbm.at[pl.ds(base, _C)], c_v, sem_g)

        def gat_drain(b):
            src_v, dst_v, a_v, b_v, c_v = b[:5]
            sem_g = b[6]
            pltpu.make_async_copy(a_hbm.at[src_v], a_v, sem_g).wait()
            pltpu.make_async_copy(b_hbm.at[dst_v], b_v, sem_g).wait()
            pltpu.make_async_copy(ce_hbm.at[pl.ds(0, _C)], c_v, sem_g).wait()

        def compute(b, half):
            dst_v, a_v, b_v, c_v = b[1], b[2], b[3], b[4]
            mo = half * _C

            def erow(i, cc):
                for r in range(2):
                    for j in range(8):
                        sl = pl.ds(j * 16, 16)
                        m_v[mo + 2 * i + r, sl] = jnp.maximum(
                            a_v[2 * i + r, sl] + b_v[2 * i + r, sl]
                            + c_v[2 * i + r, sl], 0.0)
                return cc
            lax.fori_loop(0, _C // 2, erow, 0)
            for j in (0, 16, _C - 16):
                dstc[pl.ds(mo + j, 16)] = dst_v[pl.ds(j, 16)]

        def pair_body(t, issue2=True, issue3=True):
            gat_drain(bufs[0])
            idx_drain(bufs[1])
            gat_issue(t + 1, bufs[1])
            compute(bufs[0], 0)
            if issue2:
                idx_issue(t + 2, bufs[0])
            gat_drain(bufs[1])
            if issue2:
                idx_drain(bufs[0])
                gat_issue(t + 2, bufs[0])
            compute(bufs[1], 1)
            pltpu.sync_copy(m_v, acc_sh.at[dstc], add=True)
            if issue3:
                idx_issue(t + 3, bufs[1])

        idx_issue(0, bufs[0])
        idx_drain(bufs[0])
        gat_issue(0, bufs[0])
        idx_issue(1, bufs[1])

        def zrow(i, carry):
            for j in range(8):
                m_v[i, pl.ds(j * 16, 16)] = zero
            return carry
        lax.fori_loop(0, _C, zrow, 0)
        for k in range(_RPT // _C):
            pltpu.async_copy(m_v.at[pl.ds(0, _C)],
                             acc_sh.at[pl.ds(sid * _RPT + k * _C, _C)], si0)
        for k in range(_RPT // _C):
            pltpu.make_async_copy(
                m_v.at[pl.ds(0, _C)],
                acc_sh.at[pl.ds(sid * _RPT + k * _C, _C)], si0).wait()
        plsc.subcore_barrier()

        def pair(q, carry):
            pair_body(2 * q)
            return carry
        lax.fori_loop(0, _CHUNKS // 2 - 1, pair, 0)

        pair_body(_CHUNKS - 2, issue2=False, issue3=False)

        plsc.subcore_barrier()
        pltpu.sync_copy(acc_sh.at[pl.ds(sid * _RPT, _RPT)],
                        out_hbm.at[cid, pl.ds(sid * _RPT, _RPT)])

    return _sc_edge_pass


def _enc_nodes_body(x_ref, w0, b0, w1, b1, o_ref):
    h1 = jnp.maximum(jnp.dot(x_ref[...], w0[...],
                             preferred_element_type=jnp.float32) + b0[...], 0.0)
    o_ref[...] = jnp.dot(h1, w1[...],
                         preferred_element_type=jnp.float32) + b1[...]


def _enc_edges_body(ea_ref, w0, b0, w1, b1, wee, bee, e_ref, c0_ref):
    e1 = jnp.maximum(jnp.dot(ea_ref[...], w0[...],
                             preferred_element_type=jnp.float32) + b0[...], 0.0)
    e = jnp.dot(e1, w1[...], preferred_element_type=jnp.float32) + b1[...]
    e_ref[...] = e
    c0_ref[...] = jnp.dot(e, wee[...],
                          preferred_element_type=jnp.float32) + bee[...]


def _ce_body(e_ref, wee, bee, o_ref):
    o_ref[...] = jnp.dot(e_ref[...], wee[...],
                         preferred_element_type=jnp.float32) + bee[...]


def _pre_body(h_ref, ws, wd, a_ref, b_ref):
    h = h_ref[...]
    a_ref[...] = jnp.dot(h, ws[...], preferred_element_type=jnp.float32)
    b_ref[...] = jnp.dot(h, wd[...], preferred_element_type=jnp.float32)


def _upd_body(h_ref, ag_ref, wnh, wna, bn, o_ref):
    h = h_ref[...]
    ag = ag_ref[0] + ag_ref[1]
    o_ref[...] = (h + jnp.dot(h, wnh[...], preferred_element_type=jnp.float32)
                  + jnp.dot(ag, wna[...], preferred_element_type=jnp.float32)
                  + bn[...])


def _updpre_body(h_ref, ag_ref, wnh, wna, bn, ws, wd, h_ref_o, a_ref, b_ref):
    h = h_ref[...]
    ag = ag_ref[0] + ag_ref[1]
    h2 = (h + jnp.dot(h, wnh[...], preferred_element_type=jnp.float32)
          + jnp.dot(ag, wna[...], preferred_element_type=jnp.float32)
          + bn[...])
    h_ref_o[...] = h2
    a_ref[...] = jnp.dot(h2, ws[...], preferred_element_type=jnp.float32)
    b_ref[...] = jnp.dot(h2, wd[...], preferred_element_type=jnp.float32)


def _dec_body(h_ref, w0, b0, w1, b1, w2, b2, m_ref, o_ref):
    z = jnp.maximum(jnp.dot(h_ref[...], w0[...],
                            preferred_element_type=jnp.float32) + b0[...], 0.0)
    z = jnp.maximum(jnp.dot(z, w1[...],
                            preferred_element_type=jnp.float32) + b1[...], 0.0)
    p = jnp.dot(z, w2[...], preferred_element_type=jnp.float32) + b2[...]
    o_ref[...] = p * m_ref[...]


def _full(shape):
    return pl.BlockSpec(shape, lambda i: tuple(0 for _ in shape))


def kernel(x, edge_index, edge_attr, bc_disp, bc_rot, face_mask,
           enc_W0, enc_b0, enc_W1, enc_b1,
           eenc_W0, eenc_b0, eenc_W1, eenc_b1,
           mp_We, mp_be, mp_Wn, mp_bn,
           dec_W0, dec_b0, dec_W1, dec_b1, dec_W2, dec_b2):
    f32 = jnp.float32
    blk_n = 2000
    blk_e = 1000

    xp = jnp.pad(x, ((0, 0), (0, 16 - x.shape[1])))
    enc_W0p = jnp.pad(enc_W0, ((0, 16 - enc_W0.shape[0]), (0, 0)))
    eap = jnp.pad(edge_attr, ((0, 0), (0, 16 - edge_attr.shape[1])))
    eenc_W0p = jnp.pad(eenc_W0, ((0, 16 - eenc_W0.shape[0]), (0, 0)))
    We_s = mp_We[:, :H, :]
    We_d = mp_We[:, H:2 * H, :]
    We_e = mp_We[:, 2 * H:, :]
    Wn_h = mp_Wn[:, :H, :]
    Wn_a = mp_Wn[:, H:, :]
    src1 = edge_index[0]
    dst1 = edge_index[1]
    dec_W2p = jnp.pad(dec_W2, ((0, 0), (0, 16 - OUT_DIM)))
    dec_b2p = jnp.pad(dec_b2, ((0, 16 - OUT_DIM),))
    disp_m = 1.0 - bc_disp
    rot_m = 1.0 - bc_rot
    force_m = jnp.repeat(face_mask, 3, axis=1)
    mask16 = jnp.concatenate(
        [disp_m, disp_m, rot_m, force_m, jnp.zeros((N, 1), f32)], axis=1)

    h = pl.pallas_call(
        _enc_nodes_body,
        grid=(N // blk_n,),
        in_specs=[pl.BlockSpec((blk_n, 16), lambda i: (i, 0)),
                  _full((16, H)), _full((1, H)), _full((H, H)), _full((1, H))],
        out_specs=pl.BlockSpec((blk_n, H), lambda i: (i, 0)),
        out_shape=jax.ShapeDtypeStruct((N, H), f32),
    )(xp, enc_W0p, enc_b0.reshape(1, H), enc_W1, enc_b1.reshape(1, H))

    e_enc, ce0 = pl.pallas_call(
        _enc_edges_body,
        grid=(E // blk_e,),
        in_specs=[pl.BlockSpec((blk_e, 16), lambda i: (i, 0)),
                  _full((16, H)), _full((1, H)), _full((H, H)), _full((1, H)),
                  _full((H, H)), _full((1, H))],
        out_specs=[pl.BlockSpec((blk_e, H), lambda i: (i, 0)),
                   pl.BlockSpec((blk_e, H), lambda i: (i, 0))],
        out_shape=[jax.ShapeDtypeStruct((E, H), f32),
                   jax.ShapeDtypeStruct((E, H), f32)],
    )(eap, eenc_W0p, eenc_b0.reshape(1, H), eenc_W1, eenc_b1.reshape(1, H),
      We_e[0], mp_be[0].reshape(1, H))

    ces = [ce0]
    for l in range(1, NL):
        ces.append(pl.pallas_call(
            _ce_body,
            grid=(E // blk_e,),
            in_specs=[pl.BlockSpec((blk_e, H), lambda i: (i, 0)),
                      _full((H, H)), _full((1, H))],
            out_specs=pl.BlockSpec((blk_e, H), lambda i: (i, 0)),
            out_shape=jax.ShapeDtypeStruct((E, H), f32),
        )(e_enc, We_e[l], mp_be[l].reshape(1, H)))

    a_n, b_n = pl.pallas_call(
        _pre_body,
        grid=(N // blk_n,),
        in_specs=[pl.BlockSpec((blk_n, H), lambda i: (i, 0)),
                  _full((H, H)), _full((H, H))],
        out_specs=[pl.BlockSpec((blk_n, H), lambda i: (i, 0)),
                   pl.BlockSpec((blk_n, H), lambda i: (i, 0))],
        out_shape=[jax.ShapeDtypeStruct((N, H), f32),
                   jax.ShapeDtypeStruct((N, H), f32)],
    )(h, We_s[0], We_d[0])

    for l in range(NL):
        aggp = _make_sc_edge_pass()(a_n, b_n, ces[l], src1, dst1)

        if l < NL - 1:
            h, a_n, b_n = pl.pallas_call(
                _updpre_body,
                grid=(N // blk_n,),
                in_specs=[pl.BlockSpec((blk_n, H), lambda i: (i, 0)),
                          pl.BlockSpec((_NC, blk_n, H), lambda i: (0, i, 0)),
                          _full((H, H)), _full((H, H)), _full((1, H)),
                          _full((H, H)), _full((H, H))],
                out_specs=[pl.BlockSpec((blk_n, H), lambda i: (i, 0)),
                           pl.BlockSpec((blk_n, H), lambda i: (i, 0)),
                           pl.BlockSpec((blk_n, H), lambda i: (i, 0))],
                out_shape=[jax.ShapeDtypeStruct((N, H), f32),
                           jax.ShapeDtypeStruct((N, H), f32),
                           jax.ShapeDtypeStruct((N, H), f32)],
            )(h, aggp, Wn_h[l], Wn_a[l], mp_bn[l].reshape(1, H),
              We_s[l + 1], We_d[l + 1])
        else:
            h = pl.pallas_call(
                _upd_body,
                grid=(N // blk_n,),
                in_specs=[pl.BlockSpec((blk_n, H), lambda i: (i, 0)),
                          pl.BlockSpec((_NC, blk_n, H), lambda i: (0, i, 0)),
                          _full((H, H)), _full((H, H)), _full((1, H))],
                out_specs=pl.BlockSpec((blk_n, H), lambda i: (i, 0)),
                out_shape=jax.ShapeDtypeStruct((N, H), f32),
            )(h, aggp, Wn_h[l], Wn_a[l], mp_bn[l].reshape(1, H))

    out16 = pl.pallas_call(
        _dec_body,
        grid=(N // blk_n,),
        in_specs=[pl.BlockSpec((blk_n, H), lambda i: (i, 0)),
                  _full((H, H)), _full((1, H)), _full((H, 64)), _full((1, 64)),
                  _full((64, 16)), _full((1, 16)),
                  pl.BlockSpec((blk_n, 16), lambda i: (i, 0))],
        out_specs=pl.BlockSpec((blk_n, 16), lambda i: (i, 0)),
        out_shape=jax.ShapeDtypeStruct((N, 16), f32),
    )(h, dec_W0, dec_b0.reshape(1, H), dec_W1, dec_b1.reshape(1, 64),
      dec_W2p, dec_b2p.reshape(1, 16), mask16)

    return out16[:, :OUT_DIM]


# speedup vs baseline: 1.2210x; 1.0159x over previous
import functools

import jax
import jax.numpy as jnp
from jax import lax
from jax.experimental import pallas as pl
from jax.experimental.pallas import tpu as pltpu
from jax.experimental.pallas import tpu_sc as plsc

N = 10000
E = 320000
H = 128
NL = 6
OUT_DIM = 15

_NC = 2
_NS = 16
_NW = _NC * _NS
_C = 40
_EPW = E // _NW
_CHUNKS = _EPW // _C
_NPAD = 10240
_RPT = _NPAD // _NS

@functools.cache
def _make_sc_edge_pass():
    mesh = plsc.VectorSubcoreMesh(
        core_axis_name="c", subcore_axis_name="s",
        num_cores=_NC, num_subcores=_NS)

    buf = lambda: [pltpu.VMEM((_C,), jnp.int32),
                   pltpu.VMEM((_C,), jnp.int32),
                   pltpu.VMEM((_C, H), jnp.float32),
                   pltpu.VMEM((_C, H), jnp.float32),
                   pltpu.VMEM((_C, H), jnp.float32),
                   pltpu.VMEM((_C,), jnp.int32),
                   pltpu.SemaphoreType.DMA,
                   pltpu.SemaphoreType.DMA,
                   pltpu.SemaphoreType.DMA]

    @functools.partial(
        pl.kernel,
        out_type=jax.ShapeDtypeStruct((_NC, _NPAD, H), jnp.float32),
        mesh=mesh,
        scratch_types=buf() + buf() + [
            pltpu.VMEM_SHARED((_NPAD, H), jnp.float32),
        ],
    )
    def _sc_edge_pass(a_hbm, b_hbm, ce_hbm, src_hbm, dst_hbm, out_hbm,
                      src0, dst0, a0, b0, c0, ds0, si0, sg0, ss0,
                      src1, dst1, a1, b1, c1, ds1, si1, sg1, ss1, acc_sh):
        cid = lax.axis_index("c")
        sid = lax.axis_index("s")
        wid = sid * _NC + cid
        zero = jnp.zeros((16,), jnp.float32)
        bufs = ((src0, dst0, a0, b0, c0, ds0, si0, sg0, ss0),
                (src1, dst1, a1, b1, c1, ds1, si1, sg1, ss1))


        def idx_issue(t, b):
            src_v, dst_v = b[0], b[1]
            sem_i = b[6]
            base = wid * _EPW + t * _C
            pltpu.async_copy(src_hbm.at[pl.ds(base, _C)], src_v, sem_i)
            pltpu.async_copy(dst_hbm.at[pl.ds(base, _C)], dst_v, sem_i)

        def idx_drain(b):
            src_v, dst_v = b[0], b[1]
            sem_i = b[6]
            pltpu.make_async_copy(
                src_hbm.at[pl.ds(0, _C)], src_v, sem_i).wait()
            pltpu.make_async_copy(
                dst_hbm.at[pl.ds(0, _C)], dst_v, sem_i).wait()

        def gat_issue(t, b):
            src_v, dst_v, a_v, b_v, c_v = b[:5]
            sem_g = b[7]
            base = wid * _EPW + t * _C
            pltpu.async_copy(a_hbm.at[src_v], a_v, sem_g)
            pltpu.async_copy(b_hbm.at[dst_v], b_v, sem_g)
            pltpu.async_copy(ce_hbm.at[pl.ds(base, _C)], c_v, sem_g)

        def gat_drain(b):
            src_v, dst_v, a_v, b_v, c_v = b[:5]
            sem_g = b[7]
            pltpu.make_async_copy(a_hbm.at[src_v], a_v, sem_g).wait()
            pltpu.make_async_copy(b_hbm.at[dst_v], b_v, sem_g).wait()
            pltpu.make_async_copy(ce_hbm.at[pl.ds(0, _C)], c_v, sem_g).wait()

        def compute(b):
            dst_v, a_v, b_v, c_v, dst_s = b[1], b[2], b[3], b[4], b[5]

            def erow(i, cc):
                for r in range(2):
                    for j in range(8):
                        sl = pl.ds(j * 16, 16)
                        a_v[2 * i + r, sl] = jnp.maximum(
                            a_v[2 * i + r, sl] + b_v[2 * i + r, sl]
                            + c_v[2 * i + r, sl], 0.0)
                return cc
            lax.fori_loop(0, _C // 2, erow, 0)

        def body(t, p, tail=False, issue_idx=True):
            gat_drain(bufs[p])
            if not tail:
                idx_drain(bufs[1 - p])
                gat_issue(t + 1, bufs[1 - p])
            compute(bufs[p])
            pltpu.sync_copy(bufs[p][2], acc_sh.at[bufs[p][1]], add=True)
            if not tail and issue_idx:
                idx_issue(t + 2, bufs[p])

        idx_issue(0, bufs[0])
        idx_drain(bufs[0])
        gat_issue(0, bufs[0])
        idx_issue(1, bufs[1])

        def zrow(i, carry):
            for j in range(8):
                c1[i, pl.ds(j * 16, 16)] = zero
            return carry
        lax.fori_loop(0, _C, zrow, 0)
        for k in range(_RPT // _C):
            pltpu.async_copy(c1, acc_sh.at[pl.ds(sid * _RPT + k * _C, _C)],
                             ss0)
        for k in range(_RPT // _C):
            pltpu.make_async_copy(
                c1, acc_sh.at[pl.ds(sid * _RPT + k * _C, _C)], ss0).wait()
        plsc.subcore_barrier()

        def pair(q, carry):
            body(2 * q, 0)
            body(2 * q + 1, 1)
            return carry
        lax.fori_loop(0, (_CHUNKS - 2) // 2, pair, 0)

        body(_CHUNKS - 2, 0, issue_idx=False)
        body(_CHUNKS - 1, 1, tail=True)

        plsc.subcore_barrier()
        pltpu.sync_copy(acc_sh.at[pl.ds(sid * _RPT, _RPT)],
                        out_hbm.at[cid, pl.ds(sid * _RPT, _RPT)])

    return _sc_edge_pass


def _enc_nodes_body(x_ref, w0, b0, w1, b1, o_ref):
    h1 = jnp.maximum(jnp.dot(x_ref[...], w0[...],
                             preferred_element_type=jnp.float32) + b0[...], 0.0)
    o_ref[...] = jnp.dot(h1, w1[...],
                         preferred_element_type=jnp.float32) + b1[...]


def _enc_edges_body(ea_ref, w0, b0, w1, b1, wee, bee, e_ref, c0_ref):
    e1 = jnp.maximum(jnp.dot(ea_ref[...], w0[...],
                             preferred_element_type=jnp.float32) + b0[...], 0.0)
    e = jnp.dot(e1, w1[...], preferred_element_type=jnp.float32) + b1[...]
    e_ref[...] = e
    c0_ref[...] = jnp.dot(e, wee[...],
                          preferred_element_type=jnp.float32) + bee[...]


def _ce_body(e_ref, wee, bee, o_ref):
    o_ref[...] = jnp.dot(e_ref[...], wee[...],
                         preferred_element_type=jnp.float32) + bee[...]


def _pre_body(h_ref, ws, wd, a_ref, b_ref):
    h = h_ref[...]
    a_ref[...] = jnp.dot(h, ws[...], preferred_element_type=jnp.float32)
    b_ref[...] = jnp.dot(h, wd[...], preferred_element_type=jnp.float32)


def _upd_body(h_ref, ag_ref, wnh, wna, bn, o_ref):
    h = h_ref[...]
    ag = ag_ref[0] + ag_ref[1]
    o_ref[...] = (h + jnp.dot(h, wnh[...], preferred_element_type=jnp.float32)
                  + jnp.dot(ag, wna[...], preferred_element_type=jnp.float32)
                  + bn[...])


def _updpre_body(h_ref, ag_ref, wnh, wna, bn, ws, wd, h_ref_o, a_ref, b_ref):
    h = h_ref[...]
    ag = ag_ref[0] + ag_ref[1]
    h2 = (h + jnp.dot(h, wnh[...], preferred_element_type=jnp.float32)
          + jnp.dot(ag, wna[...], preferred_element_type=jnp.float32)
          + bn[...])
    h_ref_o[...] = h2
    a_ref[...] = jnp.dot(h2, ws[...], preferred_element_type=jnp.float32)
    b_ref[...] = jnp.dot(h2, wd[...], preferred_element_type=jnp.float32)


def _dec_body(h_ref, w0, b0, w1, b1, w2, b2, m_ref, o_ref):
    z = jnp.maximum(jnp.dot(h_ref[...], w0[...],
                            preferred_element_type=jnp.float32) + b0[...], 0.0)
    z = jnp.maximum(jnp.dot(z, w1[...],
                            preferred_element_type=jnp.float32) + b1[...], 0.0)
    p = jnp.dot(z, w2[...], preferred_element_type=jnp.float32) + b2[...]
    o_ref[...] = p * m_ref[...]


def _full(shape):
    return pl.BlockSpec(shape, lambda i: tuple(0 for _ in shape))


def kernel(x, edge_index, edge_attr, bc_disp, bc_rot, face_mask,
           enc_W0, enc_b0, enc_W1, enc_b1,
           eenc_W0, eenc_b0, eenc_W1, eenc_b1,
           mp_We, mp_be, mp_Wn, mp_bn,
           dec_W0, dec_b0, dec_W1, dec_b1, dec_W2, dec_b2):
    f32 = jnp.float32
    blk_n = 2000
    blk_e = 1000

    xp = jnp.pad(x, ((0, 0), (0, 16 - x.shape[1])))
    enc_W0p = jnp.pad(enc_W0, ((0, 16 - enc_W0.shape[0]), (0, 0)))
    eap = jnp.pad(edge_attr, ((0, 0), (0, 16 - edge_attr.shape[1])))
    eenc_W0p = jnp.pad(eenc_W0, ((0, 16 - eenc_W0.shape[0]), (0, 0)))
    We_s = mp_We[:, :H, :]
    We_d = mp_We[:, H:2 * H, :]
    We_e = mp_We[:, 2 * H:, :]
    Wn_h = mp_Wn[:, :H, :]
    Wn_a = mp_Wn[:, H:, :]
    src1 = edge_index[0]
    dst1 = edge_index[1]
    dec_W2p = jnp.pad(dec_W2, ((0, 0), (0, 16 - OUT_DIM)))
    dec_b2p = jnp.pad(dec_b2, ((0, 16 - OUT_DIM),))
    disp_m = 1.0 - bc_disp
    rot_m = 1.0 - bc_rot
    force_m = jnp.repeat(face_mask, 3, axis=1)
    mask16 = jnp.concatenate(
        [disp_m, disp_m, rot_m, force_m, jnp.zeros((N, 1), f32)], axis=1)

    h = pl.pallas_call(
        _enc_nodes_body,
        grid=(N // blk_n,),
        in_specs=[pl.BlockSpec((blk_n, 16), lambda i: (i, 0)),
                  _full((16, H)), _full((1, H)), _full((H, H)), _full((1, H))],
        out_specs=pl.BlockSpec((blk_n, H), lambda i: (i, 0)),
        out_shape=jax.ShapeDtypeStruct((N, H), f32),
    )(xp, enc_W0p, enc_b0.reshape(1, H), enc_W1, enc_b1.reshape(1, H))

    e_enc, ce0 = pl.pallas_call(
        _enc_edges_body,
        grid=(E // blk_e,),
        in_specs=[pl.BlockSpec((blk_e, 16), lambda i: (i, 0)),
                  _full((16, H)), _full((1, H)), _full((H, H)), _full((1, H)),
                  _full((H, H)), _full((1, H))],
        out_specs=[pl.BlockSpec((blk_e, H), lambda i: (i, 0)),
                   pl.BlockSpec((blk_e, H), lambda i: (i, 0))],
        out_shape=[jax.ShapeDtypeStruct((E, H), f32),
                   jax.ShapeDtypeStruct((E, H), f32)],
    )(eap, eenc_W0p, eenc_b0.reshape(1, H), eenc_W1, eenc_b1.reshape(1, H),
      We_e[0], mp_be[0].reshape(1, H))

    ces = [ce0]
    for l in range(1, NL):
        ces.append(pl.pallas_call(
            _ce_body,
            grid=(E // blk_e,),
            in_specs=[pl.BlockSpec((blk_e, H), lambda i: (i, 0)),
                      _full((H, H)), _full((1, H))],
            out_specs=pl.BlockSpec((blk_e, H), lambda i: (i, 0)),
            out_shape=jax.ShapeDtypeStruct((E, H), f32),
        )(e_enc, We_e[l], mp_be[l].reshape(1, H)))

    a_n, b_n = pl.pallas_call(
        _pre_body,
        grid=(N // blk_n,),
        in_specs=[pl.BlockSpec((blk_n, H), lambda i: (i, 0)),
                  _full((H, H)), _full((H, H))],
        out_specs=[pl.BlockSpec((blk_n, H), lambda i: (i, 0)),
                   pl.BlockSpec((blk_n, H), lambda i: (i, 0))],
        out_shape=[jax.ShapeDtypeStruct((N, H), f32),
                   jax.ShapeDtypeStruct((N, H), f32)],
    )(h, We_s[0], We_d[0])

    for l in range(NL):
        aggp = _make_sc_edge_pass()(a_n, b_n, ces[l], src1, dst1)

        if l < NL - 1:
            h, a_n, b_n = pl.pallas_call(
                _updpre_body,
                grid=(N // blk_n,),
                in_specs=[pl.BlockSpec((blk_n, H), lambda i: (i, 0)),
                          pl.BlockSpec((_NC, blk_n, H), lambda i: (0, i, 0)),
                          _full((H, H)), _full((H, H)), _full((1, H)),
                          _full((H, H)), _full((H, H))],
                out_specs=[pl.BlockSpec((blk_n, H), lambda i: (i, 0)),
                           pl.BlockSpec((blk_n, H), lambda i: (i, 0)),
                           pl.BlockSpec((blk_n, H), lambda i: (i, 0))],
                out_shape=[jax.ShapeDtypeStruct((N, H), f32),
                           jax.ShapeDtypeStruct((N, H), f32),
                           jax.ShapeDtypeStruct((N, H), f32)],
            )(h, aggp, Wn_h[l], Wn_a[l], mp_bn[l].reshape(1, H),
              We_s[l + 1], We_d[l + 1])
        else:
            h = pl.pallas_call(
                _upd_body,
                grid=(N // blk_n,),
                in_specs=[pl.BlockSpec((blk_n, H), lambda i: (i, 0)),
                          pl.BlockSpec((_NC, blk_n, H), lambda i: (0, i, 0)),
                          _full((H, H)), _full((H, H)), _full((1, H))],
                out_specs=pl.BlockSpec((blk_n, H), lambda i: (i, 0)),
                out_shape=jax.ShapeDtypeStruct((N, H), f32),
            )(h, aggp, Wn_h[l], Wn_a[l], mp_bn[l].reshape(1, H))

    out16 = pl.pallas_call(
        _dec_body,
        grid=(N // blk_n,),
        in_specs=[pl.BlockSpec((blk_n, H), lambda i: (i, 0)),
                  _full((H, H)), _full((1, H)), _full((H, 64)), _full((1, 64)),
                  _full((64, 16)), _full((1, 16)),
                  pl.BlockSpec((blk_n, 16), lambda i: (i, 0))],
        out_specs=pl.BlockSpec((blk_n, 16), lambda i: (i, 0)),
        out_shape=jax.ShapeDtypeStruct((N, 16), f32),
    )(h, dec_W0, dec_b0.reshape(1, H), dec_W1, dec_b1.reshape(1, 64),
      dec_W2p, dec_b2p.reshape(1, 16), mask16)

    return out16[:, :OUT_DIM]
